# baseline (speedup 1.0000x reference)
import functools
import math

import jax
import jax.numpy as jnp
from jax import lax
from jax.experimental import pallas as pl
from jax.experimental.pallas import tpu as pltpu
from jax.experimental.pallas import tpu_sc as plsc

N = 10000
E = 320000
D = 128
D_Z = 10
DIM_KEY = 16

NC = 2
NS = 16
LANES = 16

_INV = 1.0 / math.sqrt(D * D_Z)


def _proj_body(x_ref, z_ref, wa_ref, wb_ref, s_ref, x1_ref):
    xb = x_ref[...]
    zb = z_ref[...]
    acc_a = jnp.zeros(s_ref.shape, jnp.float32)
    acc_b = jnp.zeros(x1_ref.shape, jnp.float32)
    for v in range(D_Z):
        zv = zb[:, v:v + 1]
        acc_a += zv * jnp.dot(xb, wa_ref[:, v, :], preferred_element_type=jnp.float32)
        acc_b += zv * jnp.dot(xb, wb_ref[:, v, :], preferred_element_type=jnp.float32)
    s_ref[...] = acc_a * _INV
    x1_ref[...] = acc_b * _INV


def _node_proj(x, z, wa, wb):
    BN = 1000
    grid = (N // BN,)
    return pl.pallas_call(
        _proj_body,
        grid=grid,
        in_specs=[
            pl.BlockSpec((BN, D), lambda i: (i, 0)),
            pl.BlockSpec((BN, D_Z), lambda i: (i, 0)),
            pl.BlockSpec((D, D_Z, D), lambda i: (0, 0, 0)),
            pl.BlockSpec((D, D_Z, D), lambda i: (0, 0, 0)),
        ],
        out_specs=[
            pl.BlockSpec((BN, D), lambda i: (i, 0)),
            pl.BlockSpec((BN, D), lambda i: (i, 0)),
        ],
        out_shape=[
            jax.ShapeDtypeStruct((N, D), jnp.float32),
            jax.ShapeDtypeStruct((N, D), jnp.float32),
        ],
    )(x, z, wa, wb)


def _coeff_body(emb_ref, attr_ref, t_ref, out_ref):
    out_ref[...] = attr_ref[...] * jnp.dot(
        emb_ref[...], t_ref[...], preferred_element_type=jnp.float32)


def _edge_coeff(emb, attr, t):
    BE = 8000
    grid = (E // BE,)
    return pl.pallas_call(
        _coeff_body,
        grid=grid,
        in_specs=[
            pl.BlockSpec((BE, DIM_KEY), lambda i: (i, 0)),
            pl.BlockSpec((BE, 1), lambda i: (i, 0)),
            pl.BlockSpec((DIM_KEY, D), lambda i: (0, 0)),
        ],
        out_specs=pl.BlockSpec((BE, D), lambda i: (i, 0)),
        out_shape=jax.ShapeDtypeStruct((E, D), jnp.float32),
    )(emb, attr, t)


C = 40
CHUNKS = E // C
NPAD = 10240
ROWS_PER_SUB = NPAD // NS


NW = NC * NS
PT = CHUNKS // NW
NBUF = 4


def _edge_sc_kernel(x1, src, dst, coeff, zeros):
    mesh = plsc.VectorSubcoreMesh(core_axis_name="c", subcore_axis_name="s")

    @functools.partial(
        pl.kernel,
        out_type=jax.ShapeDtypeStruct((NC, NPAD, D), jnp.float32),
        mesh=mesh,
        scratch_types=(
            [pltpu.VMEM_SHARED((NPAD, D), jnp.float32)]
            + [pltpu.VMEM((C,), jnp.int32) for _ in range(2 * NBUF)]
            + [pltpu.VMEM((C, D), jnp.float32) for _ in range(2 * NBUF)]
            + [pltpu.SemaphoreType.DMA for _ in range(3 * NBUF)]
        ),
    )
    def k(x1_hbm, src_hbm, dst_hbm, coeff_hbm, zeros_hbm, out_hbm, agg_sh, *scr):
        srcb = scr[0:NBUF]
        dstb = scr[NBUF:2 * NBUF]
        cfb = scr[2 * NBUF:3 * NBUF]
        rwb = scr[3 * NBUF:4 * NBUF]
        sin = scr[4 * NBUF:5 * NBUF]
        sg = scr[5 * NBUF:6 * NBUF]
        ss = scr[6 * NBUF:7 * NBUF]

        cid = lax.axis_index("c")
        sid = lax.axis_index("s")
        wid = sid * NC + cid
        base_c = wid * PT
        base_e = wid * PT * C

        pltpu.sync_copy(zeros_hbm.at[pl.ds(sid * ROWS_PER_SUB, ROWS_PER_SUB)],
                        agg_sh.at[pl.ds(sid * ROWS_PER_SUB, ROWS_PER_SUB)])
        plsc.subcore_barrier()

        def start_in(c, b):
            e0 = base_e + c * C
            pltpu.async_copy(src_hbm.at[pl.ds(e0, C)], srcb[b], sin[b])
            pltpu.async_copy(dst_hbm.at[pl.ds(e0, C)], dstb[b], sin[b])
            pltpu.async_copy(coeff_hbm.at[pl.ds(e0, C)], cfb[b], sin[b])

        def wait_in(c, b):
            e0 = base_e + c * C
            pltpu.make_async_copy(src_hbm.at[pl.ds(e0, C)], srcb[b], sin[b]).wait()
            pltpu.make_async_copy(dst_hbm.at[pl.ds(e0, C)], dstb[b], sin[b]).wait()
            pltpu.make_async_copy(coeff_hbm.at[pl.ds(e0, C)], cfb[b], sin[b]).wait()

        def start_gather(b):
            pltpu.async_copy(x1_hbm.at[srcb[b]], rwb[b], sg[b])

        def wait_gather(b):
            pltpu.make_async_copy(x1_hbm.at[srcb[b]], rwb[b], sg[b]).wait()

        def start_scatter(b):
            pltpu.async_copy(rwb[b], agg_sh.at[dstb[b]], ss[b], add=True)

        def wait_scatter(b):
            pltpu.make_async_copy(rwb[b], agg_sh.at[dstb[b]], ss[b]).wait()

        def mul(b):
            rows = rwb[b]
            cf = cfb[b]

            @pl.loop(0, C)
            def _mul(i):
                for j in range(D // LANES):
                    sl = (i, pl.ds(j * LANES, LANES))
                    rows[sl] = rows[sl] * cf[sl]

        def step(c, b):
            bp1 = (b + 1) % NBUF
            bp2 = (b + 2) % NBUF

            @pl.when(c >= 2)
            def _():
                wait_scatter(bp2)

            @pl.when(c + 2 <= PT - 1)
            def _():
                start_in(c + 2, bp2)

            @pl.when(c + 1 <= PT - 1)
            def _():
                wait_in(c + 1, bp1)
                start_gather(bp1)

            wait_gather(b)
            mul(b)
            start_scatter(b)

        start_in(0, 0)
        start_in(1, 1)
        wait_in(0, 0)
        start_gather(0)

        @pl.loop(0, PT // NBUF)
        def _main(kk):
            for b in range(NBUF):
                step(kk * NBUF + b, b)

        for c in range(PT - PT % NBUF, PT):
            step(c, c % NBUF)

        wait_scatter((PT - 2) % NBUF)
        wait_scatter((PT - 1) % NBUF)

        plsc.subcore_barrier()
        pltpu.sync_copy(agg_sh.at[pl.ds(sid * ROWS_PER_SUB, ROWS_PER_SUB)],
                        out_hbm.at[cid, pl.ds(sid * ROWS_PER_SUB, ROWS_PER_SUB)])

    return k(x1, src, dst, coeff, zeros)


def _final_body(p_ref, z_ref, w2_ref, s_ref, out_ref):
    agg = p_ref[0] + p_ref[1]
    zb = z_ref[...]
    acc = jnp.zeros(out_ref.shape, jnp.float32)
    for v in range(D_Z):
        acc += zb[:, v:v + 1] * jnp.dot(agg, w2_ref[:, v, :], preferred_element_type=jnp.float32)
    out_ref[...] = s_ref[...] + acc * (_INV / 10.0)


def _final(partials, z, w2, s):
    BN = 1000
    grid = (N // BN,)
    return pl.pallas_call(
        _final_body,
        grid=grid,
        in_specs=[
            pl.BlockSpec((NC, BN, D), lambda i: (0, i, 0)),
            pl.BlockSpec((BN, D_Z), lambda i: (i, 0)),
            pl.BlockSpec((D, D_Z, D), lambda i: (0, 0, 0)),
            pl.BlockSpec((BN, D), lambda i: (i, 0)),
        ],
        out_specs=pl.BlockSpec((BN, D), lambda i: (i, 0)),
        out_shape=jax.ShapeDtypeStruct((N, D), jnp.float32),
    )(partials, z, w2, s)


def kernel(x, z, edge_src, edge_dst, edge_len_emb, edge_attr, W_si, W_lin1, tp_weight, W_lin2):
    s, x1 = _node_proj(x, z, W_si, W_lin1)
    coeff = _edge_coeff(edge_len_emb, edge_attr, tp_weight)
    zeros = jnp.zeros((NPAD, D), jnp.float32)
    partials = _edge_sc_kernel(x1, edge_src, edge_dst, coeff, zeros)
    return _final(partials, z, W_lin2, s)


# speedup vs baseline: 1.6862x; 1.6862x over previous
import functools
import math

import jax
import jax.numpy as jnp
from jax import lax
from jax.experimental import pallas as pl
from jax.experimental.pallas import tpu as pltpu
from jax.experimental.pallas import tpu_sc as plsc

N = 10000
E = 320000
D = 128
D_Z = 10
DIM_KEY = 16

NC = 2
NS = 16
LANES = 16

_INV = 1.0 / math.sqrt(D * D_Z)


def _proj_body(x_ref, z_ref, wa_ref, wb_ref, s_ref, x1_ref):
    xb = x_ref[...]
    zb = z_ref[...]
    acc_a = jnp.zeros(s_ref.shape, jnp.float32)
    acc_b = jnp.zeros(x1_ref.shape, jnp.float32)
    for v in range(D_Z):
        zv = zb[:, v:v + 1]
        acc_a += zv * jnp.dot(xb, wa_ref[:, v, :], preferred_element_type=jnp.float32)
        acc_b += zv * jnp.dot(xb, wb_ref[:, v, :], preferred_element_type=jnp.float32)
    s_ref[...] = acc_a * _INV
    x1_ref[...] = acc_b * _INV


def _node_proj(x, z, wa, wb):
    BN = 1000
    grid = (N // BN,)
    return pl.pallas_call(
        _proj_body,
        grid=grid,
        in_specs=[
            pl.BlockSpec((BN, D), lambda i: (i, 0)),
            pl.BlockSpec((BN, D_Z), lambda i: (i, 0)),
            pl.BlockSpec((D, D_Z, D), lambda i: (0, 0, 0)),
            pl.BlockSpec((D, D_Z, D), lambda i: (0, 0, 0)),
        ],
        out_specs=[
            pl.BlockSpec((BN, D), lambda i: (i, 0)),
            pl.BlockSpec((BN, D), lambda i: (i, 0)),
        ],
        out_shape=[
            jax.ShapeDtypeStruct((N, D), jnp.float32),
            jax.ShapeDtypeStruct((N, D), jnp.float32),
        ],
    )(x, z, wa, wb)


def _coeff_body(embt_ref, attr_ref, t_ref, out_ref):
    scaled = embt_ref[...] * attr_ref[...]
    out_ref[...] = jax.lax.dot_general(
        scaled, t_ref[...], (((0,), (0,)), ((), ())),
        preferred_element_type=jnp.float32)


def _edge_coeff(embt, attr_row, t):
    BE = 12800
    grid = (E // BE,)
    return pl.pallas_call(
        _coeff_body,
        grid=grid,
        in_specs=[
            pl.BlockSpec((DIM_KEY, BE), lambda i: (0, i)),
            pl.BlockSpec((1, BE), lambda i: (0, i)),
            pl.BlockSpec((DIM_KEY, D), lambda i: (0, 0)),
        ],
        out_specs=pl.BlockSpec((BE, D), lambda i: (i, 0)),
        out_shape=jax.ShapeDtypeStruct((E, D), jnp.float32),
    )(embt, attr_row, t)


C = 40
CHUNKS = E // C
NPAD = 10240
ROWS_PER_SUB = NPAD // NS


NW = NC * NS
PT = CHUNKS // NW
NBUF = 4


def _edge_sc_kernel(x1, src, dst, coeff, zeros):
    mesh = plsc.VectorSubcoreMesh(core_axis_name="c", subcore_axis_name="s")

    @functools.partial(
        pl.kernel,
        out_type=jax.ShapeDtypeStruct((NC, NPAD, D), jnp.float32),
        mesh=mesh,
        scratch_types=(
            [pltpu.VMEM_SHARED((NPAD, D), jnp.float32)]
            + [pltpu.VMEM((C,), jnp.int32) for _ in range(2 * NBUF)]
            + [pltpu.VMEM((C, D), jnp.float32) for _ in range(2 * NBUF)]
            + [pltpu.SemaphoreType.DMA for _ in range(3 * NBUF)]
        ),
    )
    def k(x1_hbm, src_hbm, dst_hbm, coeff_hbm, zeros_hbm, out_hbm, agg_sh, *scr):
        srcb = scr[0:NBUF]
        dstb = scr[NBUF:2 * NBUF]
        cfb = scr[2 * NBUF:3 * NBUF]
        rwb = scr[3 * NBUF:4 * NBUF]
        sin = scr[4 * NBUF:5 * NBUF]
        sg = scr[5 * NBUF:6 * NBUF]
        ss = scr[6 * NBUF:7 * NBUF]

        cid = lax.axis_index("c")
        sid = lax.axis_index("s")
        wid = sid * NC + cid
        base_c = wid * PT
        base_e = wid * PT * C

        pltpu.sync_copy(zeros_hbm.at[pl.ds(sid * ROWS_PER_SUB, ROWS_PER_SUB)],
                        agg_sh.at[pl.ds(sid * ROWS_PER_SUB, ROWS_PER_SUB)])
        plsc.subcore_barrier()

        def start_in(c, b):
            e0 = base_e + c * C
            pltpu.async_copy(src_hbm.at[pl.ds(e0, C)], srcb[b], sin[b])
            pltpu.async_copy(dst_hbm.at[pl.ds(e0, C)], dstb[b], sin[b])
            pltpu.async_copy(coeff_hbm.at[pl.ds(e0, C)], cfb[b], sin[b])

        def wait_in(c, b):
            e0 = base_e + c * C
            pltpu.make_async_copy(src_hbm.at[pl.ds(e0, C)], srcb[b], sin[b]).wait()
            pltpu.make_async_copy(dst_hbm.at[pl.ds(e0, C)], dstb[b], sin[b]).wait()
            pltpu.make_async_copy(coeff_hbm.at[pl.ds(e0, C)], cfb[b], sin[b]).wait()

        def start_gather(b):
            pltpu.async_copy(x1_hbm.at[srcb[b]], rwb[b], sg[b])

        def wait_gather(b):
            pltpu.make_async_copy(x1_hbm.at[srcb[b]], rwb[b], sg[b]).wait()

        def start_scatter(b):
            pltpu.async_copy(rwb[b], agg_sh.at[dstb[b]], ss[b], add=True)

        def wait_scatter(b):
            pltpu.make_async_copy(rwb[b], agg_sh.at[dstb[b]], ss[b]).wait()

        def mul(b):
            rows = rwb[b]
            cf = cfb[b]

            @pl.loop(0, C)
            def _mul(i):
                for j in range(D // LANES):
                    sl = (i, pl.ds(j * LANES, LANES))
                    rows[sl] = rows[sl] * cf[sl]

        def step(c, b):
            bp1 = (b + 1) % NBUF
            bp2 = (b + 2) % NBUF

            @pl.when(c >= 2)
            def _():
                wait_scatter(bp2)

            @pl.when(c + 2 <= PT - 1)
            def _():
                start_in(c + 2, bp2)

            @pl.when(c + 1 <= PT - 1)
            def _():
                wait_in(c + 1, bp1)
                start_gather(bp1)

            wait_gather(b)
            mul(b)
            start_scatter(b)

        start_in(0, 0)
        start_in(1, 1)
        wait_in(0, 0)
        start_gather(0)

        @pl.loop(0, PT // NBUF)
        def _main(kk):
            for b in range(NBUF):
                step(kk * NBUF + b, b)

        for c in range(PT - PT % NBUF, PT):
            step(c, c % NBUF)

        wait_scatter((PT - 2) % NBUF)
        wait_scatter((PT - 1) % NBUF)

        plsc.subcore_barrier()
        pltpu.sync_copy(agg_sh.at[pl.ds(sid * ROWS_PER_SUB, ROWS_PER_SUB)],
                        out_hbm.at[cid, pl.ds(sid * ROWS_PER_SUB, ROWS_PER_SUB)])

    return k(x1, src, dst, coeff, zeros)


def _final_body(p_ref, z_ref, w2_ref, s_ref, out_ref):
    agg = p_ref[0] + p_ref[1]
    zb = z_ref[...]
    acc = jnp.zeros(out_ref.shape, jnp.float32)
    for v in range(D_Z):
        acc += zb[:, v:v + 1] * jnp.dot(agg, w2_ref[:, v, :], preferred_element_type=jnp.float32)
    out_ref[...] = s_ref[...] + acc * (_INV / 10.0)


def _final(partials, z, w2, s):
    BN = 1000
    grid = (N // BN,)
    return pl.pallas_call(
        _final_body,
        grid=grid,
        in_specs=[
            pl.BlockSpec((NC, BN, D), lambda i: (0, i, 0)),
            pl.BlockSpec((BN, D_Z), lambda i: (i, 0)),
            pl.BlockSpec((D, D_Z, D), lambda i: (0, 0, 0)),
            pl.BlockSpec((BN, D), lambda i: (i, 0)),
        ],
        out_specs=pl.BlockSpec((BN, D), lambda i: (i, 0)),
        out_shape=jax.ShapeDtypeStruct((N, D), jnp.float32),
    )(partials, z, w2, s)


def kernel(x, z, edge_src, edge_dst, edge_len_emb, edge_attr, W_si, W_lin1, tp_weight, W_lin2):
    s, x1 = _node_proj(x, z, W_si, W_lin1)
    coeff = _edge_coeff(edge_len_emb.T, edge_attr.T, tp_weight)
    zeros = jnp.zeros((NPAD, D), jnp.float32)
    partials = _edge_sc_kernel(x1, edge_src, edge_dst, coeff, zeros)
    return _final(partials, z, W_lin2, s)
